# Initial kernel scaffold; baseline (speedup 1.0000x reference)
import jax
import jax.numpy as jnp
from jax.experimental import pallas as pl


def kernel(hidden_states, output_expert_usage_loss, pad_mask, gate_w, w1_in, w1_out, w2_in, w2_out):
    raise NotImplementedError("write your pallas kernel here")


# speedup vs baseline: 1.2343x; 1.2343x over previous
import functools

import jax
import jax.numpy as jnp
from jax.experimental import pallas as pl
from jax.experimental.pallas import tpu as pltpu

_TB = 512
_D = 1024
_E1 = 1024
_E2 = 2048


def _moe_block(x_ref, gw_ref, w1i_ref, w1o_ref, w2i_ref, w2o_ref,
               out_ref, logits_ref):
    x = x_ref[...]
    xb = x.astype(jnp.bfloat16)

    lp = jnp.dot(xb, gw_ref[...], preferred_element_type=jnp.float32)
    logits_ref[...] = lp[:, :3]
    l0, l1, l2 = lp[:, 0:1], lp[:, 1:2], lp[:, 2:3]
    m = jnp.maximum(jnp.maximum(l0, l1), l2)
    e0 = jnp.exp(l0 - m)
    e1 = jnp.exp(l1 - m)
    e2 = jnp.exp(l2 - m)
    s = e0 + e1 + e2
    p0, p1, p2 = e0 / s, e1 / s, e2 / s
    pmin = jnp.minimum(jnp.minimum(p0, p1), p2)
    drop2 = p2 <= pmin
    drop1 = jnp.logical_and(jnp.logical_not(drop2), p1 <= pmin)
    drop0 = jnp.logical_not(jnp.logical_or(drop1, drop2))
    w0 = jnp.where(drop0, 0.0, p0)
    w1 = jnp.where(drop1, 0.0, p1)
    w2 = jnp.where(drop2, 0.0, p2)
    inv = 1.0 / (w0 + w1 + w2)

    xp = jnp.dot(xb, w1i_ref[...], preferred_element_type=jnp.float32)
    a, b = xp[:, :_E1], xp[:, _E1:]
    g = (a * jax.nn.sigmoid(a) * b).astype(jnp.bfloat16)
    o1 = jnp.dot(g, w1o_ref[...], preferred_element_type=jnp.float32)

    xp2 = jnp.dot(xb, w2i_ref[...], preferred_element_type=jnp.float32)
    a2, b2 = xp2[:, :_E2], xp2[:, _E2:]
    g2 = (a2 * jax.nn.sigmoid(a2) * b2).astype(jnp.bfloat16)
    o2 = jnp.dot(g2, w2o_ref[...], preferred_element_type=jnp.float32)

    out_ref[...] = (w0 * x + w1 * o1 + w2 * o2) * inv


@functools.partial(jax.jit, static_argnums=())
def kernel(hidden_states, output_expert_usage_loss, pad_mask, gate_w,
           w1_in, w1_out, w2_in, w2_out):
    B, S, D = hidden_states.shape
    T = B * S
    h = hidden_states.reshape(T, D)
    gw = jnp.zeros((D, 128), gate_w.dtype).at[:, :3].set(gate_w)
    gw = gw.astype(jnp.bfloat16)
    w1i = w1_in.astype(jnp.bfloat16)
    w1o = w1_out.astype(jnp.bfloat16)
    w2i = w2_in.astype(jnp.bfloat16)
    w2o = w2_out.astype(jnp.bfloat16)

    grid = (T // _TB,)
    out, logits = pl.pallas_call(
        _moe_block,
        grid=grid,
        in_specs=[
            pl.BlockSpec((_TB, D), lambda i: (i, 0)),
            pl.BlockSpec((D, 128), lambda i: (0, 0)),
            pl.BlockSpec((D, 2 * _E1), lambda i: (0, 0)),
            pl.BlockSpec((_E1, D), lambda i: (0, 0)),
            pl.BlockSpec((D, 2 * _E2), lambda i: (0, 0)),
            pl.BlockSpec((_E2, D), lambda i: (0, 0)),
        ],
        out_specs=[
            pl.BlockSpec((_TB, D), lambda i: (i, 0)),
            pl.BlockSpec((_TB, 3), lambda i: (i, 0)),
        ],
        out_shape=[
            jax.ShapeDtypeStruct((T, D), jnp.float32),
            jax.ShapeDtypeStruct((T, 3), jnp.float32),
        ],
        compiler_params=pltpu.CompilerParams(
            dimension_semantics=("arbitrary",),
        ),
    )(h, gw, w1i, w1o, w2i, w2o)

    return out.reshape(B, S, D), logits


# speedup vs baseline: 1.2466x; 1.0099x over previous
import functools

import jax
import jax.numpy as jnp
from jax.experimental import pallas as pl
from jax.experimental.pallas import tpu as pltpu

_TB = 1024
_D = 1024
_E1 = 1024
_E2 = 2048


def _moe_block(x_ref, gw_ref, w1i_ref, w1o_ref, w2i_ref, w2o_ref,
               out_ref, logits_ref):
    x = x_ref[...]
    xb = x.astype(jnp.bfloat16)

    lp = jnp.dot(xb, gw_ref[...], preferred_element_type=jnp.float32)
    logits_ref[...] = lp[:, :3]
    l0, l1, l2 = lp[:, 0:1], lp[:, 1:2], lp[:, 2:3]
    m = jnp.maximum(jnp.maximum(l0, l1), l2)
    e0 = jnp.exp(l0 - m)
    e1 = jnp.exp(l1 - m)
    e2 = jnp.exp(l2 - m)
    s = e0 + e1 + e2
    p0, p1, p2 = e0 / s, e1 / s, e2 / s
    pmin = jnp.minimum(jnp.minimum(p0, p1), p2)
    drop2 = p2 <= pmin
    drop1 = jnp.logical_and(jnp.logical_not(drop2), p1 <= pmin)
    drop0 = jnp.logical_not(jnp.logical_or(drop1, drop2))
    w0 = jnp.where(drop0, 0.0, p0)
    w1 = jnp.where(drop1, 0.0, p1)
    w2 = jnp.where(drop2, 0.0, p2)
    inv = 1.0 / (w0 + w1 + w2)

    xp = jnp.dot(xb, w1i_ref[...], preferred_element_type=jnp.float32)
    a, b = xp[:, :_E1], xp[:, _E1:]
    g = (a * jax.nn.sigmoid(a) * b).astype(jnp.bfloat16)
    o1 = jnp.dot(g, w1o_ref[...], preferred_element_type=jnp.float32)

    xp2 = jnp.dot(xb, w2i_ref[...], preferred_element_type=jnp.float32)
    a2, b2 = xp2[:, :_E2], xp2[:, _E2:]
    g2 = (a2 * jax.nn.sigmoid(a2) * b2).astype(jnp.bfloat16)
    o2 = jnp.dot(g2, w2o_ref[...], preferred_element_type=jnp.float32)

    out_ref[...] = (w0 * x + w1 * o1 + w2 * o2) * inv


@functools.partial(jax.jit, static_argnums=())
def kernel(hidden_states, output_expert_usage_loss, pad_mask, gate_w,
           w1_in, w1_out, w2_in, w2_out):
    B, S, D = hidden_states.shape
    T = B * S
    h = hidden_states.reshape(T, D)
    gw = jnp.zeros((D, 128), gate_w.dtype).at[:, :3].set(gate_w)
    gw = gw.astype(jnp.bfloat16)
    w1i = w1_in.astype(jnp.bfloat16)
    w1o = w1_out.astype(jnp.bfloat16)
    w2i = w2_in.astype(jnp.bfloat16)
    w2o = w2_out.astype(jnp.bfloat16)

    grid = (T // _TB,)
    out, logits = pl.pallas_call(
        _moe_block,
        grid=grid,
        in_specs=[
            pl.BlockSpec((_TB, D), lambda i: (i, 0)),
            pl.BlockSpec((D, 128), lambda i: (0, 0)),
            pl.BlockSpec((D, 2 * _E1), lambda i: (0, 0)),
            pl.BlockSpec((_E1, D), lambda i: (0, 0)),
            pl.BlockSpec((D, 2 * _E2), lambda i: (0, 0)),
            pl.BlockSpec((_E2, D), lambda i: (0, 0)),
        ],
        out_specs=[
            pl.BlockSpec((_TB, D), lambda i: (i, 0)),
            pl.BlockSpec((_TB, 3), lambda i: (i, 0)),
        ],
        out_shape=[
            jax.ShapeDtypeStruct((T, D), jnp.float32),
            jax.ShapeDtypeStruct((T, 3), jnp.float32),
        ],
        compiler_params=pltpu.CompilerParams(
            dimension_semantics=("arbitrary",),
        ),
    )(h, gw, w1i, w1o, w2i, w2o)

    return out.reshape(B, S, D), logits
